# baseline (speedup 1.0000x reference)
import functools
import jax
import jax.numpy as jnp
from jax import lax
from jax.experimental import pallas as pl
from jax.experimental.pallas import tpu as pltpu
from jax.experimental.pallas import tpu_sc as plsc

_B, _R, _C, _D = 64, 32, 32, 768
_NROWS = _B * _R * _C
_NDROP = max(1, int(_R * _C * 0.25))
_BLK = 2048
_NW = 32
_KPW = _NDROP // _NW


def _flat_drop_ids():
    b_rand = jax.random.randint(jax.random.key(1), (_NDROP,), 0, _B)
    r_rand = jax.random.randint(jax.random.key(2), (_NDROP,), 0, _R)
    c_rand = jax.random.randint(jax.random.key(3), (_NDROP,), 0, _C)
    return (b_rand * _R + r_rand) * _C + c_rand


def _copy_body(x_ref, o_ref):
    o_ref[...] = x_ref[...]


def _tc_copy(x2):
    return pl.pallas_call(
        _copy_body,
        grid=(_NROWS // _BLK,),
        in_specs=[pl.BlockSpec((_BLK, _D), lambda i: (i, 0))],
        out_specs=pl.BlockSpec((_BLK, _D), lambda i: (i, 0)),
        out_shape=jax.ShapeDtypeStruct((_NROWS, _D), x2.dtype),
        compiler_params=pltpu.CompilerParams(
            dimension_semantics=("arbitrary",),
        ),
    )(x2)


_mesh = plsc.VectorSubcoreMesh(
    core_axis_name="c", subcore_axis_name="s", num_cores=2, num_subcores=16
)


@functools.partial(
    pl.kernel,
    mesh=_mesh,
    scratch_types=[
        pltpu.VMEM((_KPW,), jnp.int32),
        pltpu.VMEM((_KPW, _D), jnp.float32),
        pltpu.SemaphoreType.DMA,
    ],
)
def _sc_scatter(idx_hbm, tok_hbm, out_ref, idx_v, rows_v, sem):
    wid = lax.axis_index("s") * 2 + lax.axis_index("c")
    base = wid * _KPW
    pltpu.sync_copy(idx_hbm.at[pl.ds(base, _KPW)], idx_v)
    for j in range(_KPW):
        pltpu.sync_copy(tok_hbm, rows_v.at[j])
    pltpu.async_copy(rows_v, out_ref.at[idx_v], sem).wait()


def kernel(x, mask_token):
    x2 = x.reshape(_NROWS, _D)
    idx = _flat_drop_ids()
    copied = _tc_copy(x2)
    ref = jax.new_ref(copied)
    _sc_scatter(idx, mask_token, ref)
    return jax.freeze(ref).reshape(_B, _R, _C, _D)


# speedup vs baseline: 1.1940x; 1.1940x over previous
import jax
import jax.numpy as jnp
from jax.experimental import pallas as pl
from jax.experimental.pallas import tpu as pltpu

_B, _R, _C, _D = 64, 32, 32, 768
_NROWS = _B * _R * _C
_NDROP = max(1, int(_R * _C * 0.25))
_BLK = 4096


def _flat_drop_ids():
    b_rand = jax.random.randint(jax.random.key(1), (_NDROP,), 0, _B)
    r_rand = jax.random.randint(jax.random.key(2), (_NDROP,), 0, _R)
    c_rand = jax.random.randint(jax.random.key(3), (_NDROP,), 0, _C)
    return (b_rand * _R + r_rand) * _C + c_rand


def _masked_copy_body(idx_ref, x_ref, tok_ref, o_ref):
    row0 = pl.program_id(0) * _BLK
    rows = jax.lax.broadcasted_iota(jnp.int32, (_BLK, 1), 0) + row0
    hit = jnp.any(rows == idx_ref[0, :][None, :], axis=-1, keepdims=True)
    o_ref[...] = jnp.where(hit, tok_ref[...], x_ref[...])


def kernel(x, mask_token):
    x2 = x.reshape(_NROWS, _D)
    tok = mask_token.reshape(1, _D)
    idx = _flat_drop_ids().reshape(1, _NDROP)
    out = pl.pallas_call(
        _masked_copy_body,
        grid=(_NROWS // _BLK,),
        in_specs=[
            pl.BlockSpec((1, _NDROP), lambda i: (0, 0)),
            pl.BlockSpec((_BLK, _D), lambda i: (i, 0)),
            pl.BlockSpec((1, _D), lambda i: (0, 0)),
        ],
        out_specs=pl.BlockSpec((_BLK, _D), lambda i: (i, 0)),
        out_shape=jax.ShapeDtypeStruct((_NROWS, _D), x.dtype),
        compiler_params=pltpu.CompilerParams(
            dimension_semantics=("arbitrary",),
        ),
    )(idx, x2, tok)
    return out.reshape(_B, _R, _C, _D)
